# speedup vs baseline: 2.2312x; 2.2312x over previous
import functools

import jax
import jax.numpy as jnp
from jax.experimental import pallas as pl
from jax.experimental.pallas import tpu as pltpu

N_NODES = 10000
EMB = 128
N_REL = 4
N_EDGES = 320000
SEG = N_REL * N_NODES

BN = 2000


def _dense_body(final_avg, x_ref, bcat_ref, root_ref, wstack_ref, b_ref,
                x0_ref, out_ref):
    x = x_ref[...]
    out = jnp.dot(x, root_ref[...], preferred_element_type=jnp.float32)
    out += jnp.dot(bcat_ref[...], wstack_ref[...],
                   preferred_element_type=jnp.float32)
    out += b_ref[...]
    if final_avg:
        out_ref[...] = (x0_ref[...] + x + out) * (1.0 / 3.0)
    else:
        out_ref[...] = out


def _dense_layer(x, bcat, root, wstack, b, x0, final_avg):
    grid = (N_NODES // BN,)
    return pl.pallas_call(
        functools.partial(_dense_body, final_avg),
        grid=grid,
        in_specs=[
            pl.BlockSpec((BN, EMB), lambda i: (i, 0)),
            pl.BlockSpec((BN, N_REL * EMB), lambda i: (i, 0)),
            pl.BlockSpec((EMB, EMB), lambda i: (0, 0)),
            pl.BlockSpec((N_REL * EMB, EMB), lambda i: (0, 0)),
            pl.BlockSpec((1, EMB), lambda i: (0, 0)),
            pl.BlockSpec((BN, EMB), lambda i: (i, 0)),
        ],
        out_specs=pl.BlockSpec((BN, EMB), lambda i: (i, 0)),
        out_shape=jax.ShapeDtypeStruct((N_NODES, EMB), jnp.float32),
    )(x, bcat, root, wstack, b, x0)


def kernel(edge_index_mp, edge_type, emb, w0, root0, b0, w1, root1, b1):
    src = edge_index_mp[0]
    dst = edge_index_mp[1]
    seg = edge_type * N_NODES + dst

    ones = jnp.ones((N_EDGES,), jnp.float32)
    cnt = jax.ops.segment_sum(ones, seg, num_segments=SEG)
    inv = 1.0 / jnp.clip(cnt, 1.0, None)

    def layer(x, w, root, b, x0, final_avg):
        xs = jnp.take(x, src, axis=0)
        acc = jax.ops.segment_sum(xs, seg, num_segments=SEG)
        bmat = acc * inv[:, None]
        bcat = bmat.reshape(N_REL, N_NODES, EMB).transpose(1, 0, 2)
        bcat = bcat.reshape(N_NODES, N_REL * EMB)
        return _dense_layer(x, bcat, root, w.reshape(N_REL * EMB, EMB),
                            b.reshape(1, EMB), x0, final_avg)

    x1 = layer(emb, w0, root0, b0, emb, False)
    x2f = layer(x1, w1, root1, b1, emb, True)
    return x2f


# speedup vs baseline: 14.6817x; 6.5801x over previous
import functools

import jax
import jax.numpy as jnp
from jax import lax
from jax.experimental import pallas as pl
from jax.experimental.pallas import tpu as pltpu
from jax.experimental.pallas import tpu_sc as plsc

N_NODES = 10000
EMB = 128
N_REL = 4
N_EDGES = 320000
SEG = N_REL * N_NODES

CW = 32
NCHUNK = EMB // CW
G = 80
NG = 25
SB = G * NG
RING = 8
TILE_EDGES = N_EDGES // 16
NSUPER = TILE_EDGES // SB
ROWS_PER_TILE = SEG // 16
ZR = 100

BN = 2000


def _sc_body(src_hbm, dst_hbm, et_hbm, x2d_hbm, acc_out,
             acc_s, srcv, dstv, etv, gidx, segb, rows, zbuf, gsem):
    c = lax.axis_index("c")
    s = lax.axis_index("s")
    tile_base = s * TILE_EDGES
    acc_base = s * ROWS_PER_TILE

    def zfill(i, carry):
        zbuf[i, pl.ds(0, 16)] = jnp.zeros((16,), jnp.float32)
        zbuf[i, pl.ds(16, 16)] = jnp.zeros((16,), jnp.float32)
        return carry
    lax.fori_loop(0, ZR, zfill, 0)

    for p in range(2):
        chunk = c * 2 + p

        def zero_body(i, carry):
            pltpu.sync_copy(zbuf, acc_s.at[pl.ds(acc_base + i * ZR, ZR)])
            return carry
        lax.fori_loop(0, ROWS_PER_TILE // ZR, zero_body, 0)
        plsc.subcore_barrier()

        def super_body(sb, carry):
            base = tile_base + sb * SB
            pltpu.sync_copy(src_hbm.at[pl.ds(base, SB)], srcv)
            pltpu.sync_copy(dst_hbm.at[pl.ds(base, SB)], dstv)
            pltpu.sync_copy(et_hbm.at[pl.ds(base, SB)], etv)

            def idx_body(g, carry2):
                o = g * G
                for k in range(G // 16):
                    sv = srcv[pl.ds(o + 16 * k, 16)]
                    dv = dstv[pl.ds(o + 16 * k, 16)]
                    ev = etv[pl.ds(o + 16 * k, 16)]
                    gidx[g, pl.ds(16 * k, 16)] = sv * NCHUNK + chunk
                    segb[g, pl.ds(16 * k, 16)] = ev * N_NODES + dv
                return carry2
            lax.fori_loop(0, NG, idx_body, 0)

            def fire(g, carry2):
                pltpu.make_async_copy(
                    x2d_hbm.at[gidx.at[g]], rows.at[g], gsem).start()
                return carry2
            lax.fori_loop(0, RING, fire, 0)

            def drain_scat(g, carry2):
                slot = lax.rem(g, RING)
                pltpu.make_async_copy(
                    x2d_hbm.at[gidx.at[g]], rows.at[slot], gsem).wait()
                pltpu.sync_copy(rows.at[slot], acc_s.at[segb.at[g]],
                                add=True)

                @pl.when(g + RING < NG)
                def _():
                    nxt = g + RING
                    pltpu.make_async_copy(
                        x2d_hbm.at[gidx.at[nxt]],
                        rows.at[lax.rem(nxt, RING)], gsem).start()
                return carry2
            lax.fori_loop(0, NG, drain_scat, 0)
            return carry
        lax.fori_loop(0, NSUPER, super_body, 0)

        plsc.subcore_barrier()
        pltpu.sync_copy(
            acc_s.at[pl.ds(acc_base, ROWS_PER_TILE)],
            acc_out.at[chunk, pl.ds(acc_base, ROWS_PER_TILE)])
        plsc.subcore_barrier()


def _make_sc_segsum():
    mesh = plsc.VectorSubcoreMesh(core_axis_name="c", subcore_axis_name="s")
    scratch = (
        pltpu.VMEM_SHARED((SEG, CW), jnp.float32),
        pltpu.VMEM((SB,), jnp.int32),
        pltpu.VMEM((SB,), jnp.int32),
        pltpu.VMEM((SB,), jnp.int32),
        pltpu.VMEM((NG, G), jnp.int32),
        pltpu.VMEM((NG, G), jnp.int32),
        pltpu.VMEM((RING, G, CW), jnp.float32),
        pltpu.VMEM((ZR, CW), jnp.float32),
        pltpu.SemaphoreType.DMA,
    )
    return pl.kernel(
        _sc_body, mesh=mesh,
        out_type=(jax.ShapeDtypeStruct((NCHUNK, SEG, CW), jnp.float32),),
        scratch_types=scratch,
        compiler_params=pltpu.CompilerParams(use_tc_tiling_on_sc=False))


def _cnt_body(dst_hbm, et_hbm, cnt_out,
              cnt_s, dstv, etv, segb, ones, zbuf16):
    c = lax.axis_index("c")
    s = lax.axis_index("s")
    tile_base = s * TILE_EDGES
    acc_base = s * ROWS_PER_TILE

    def zfill(i, carry):
        zbuf16[i, pl.ds(0, 16)] = jnp.zeros((16,), jnp.float32)
        return carry
    lax.fori_loop(0, ZR, zfill, 0)

    def ofill(i, carry):
        ones[i, pl.ds(0, 16)] = jnp.ones((16,), jnp.float32)
        return carry
    lax.fori_loop(0, G, ofill, 0)

    @pl.when(c == 0)
    def _():
        def zero_body(i, carry):
            pltpu.sync_copy(zbuf16, cnt_s.at[pl.ds(acc_base + i * ZR, ZR)])
            return carry
        lax.fori_loop(0, ROWS_PER_TILE // ZR, zero_body, 0)
    plsc.subcore_barrier()

    @pl.when(c == 0)
    def _():
        def super_body(sb, carry):
            base = tile_base + sb * SB
            pltpu.sync_copy(dst_hbm.at[pl.ds(base, SB)], dstv)
            pltpu.sync_copy(et_hbm.at[pl.ds(base, SB)], etv)

            def idx_body(g, carry2):
                o = g * G
                for k in range(G // 16):
                    dv = dstv[pl.ds(o + 16 * k, 16)]
                    ev = etv[pl.ds(o + 16 * k, 16)]
                    segb[g, pl.ds(16 * k, 16)] = ev * N_NODES + dv
                return carry2
            lax.fori_loop(0, NG, idx_body, 0)

            def scat_body(g, carry2):
                pltpu.sync_copy(ones, cnt_s.at[segb.at[g]], add=True)
                return carry2
            lax.fori_loop(0, NG, scat_body, 0)
            return carry
        lax.fori_loop(0, NSUPER, super_body, 0)
    plsc.subcore_barrier()

    @pl.when(c == 0)
    def _():
        pltpu.sync_copy(cnt_s.at[pl.ds(acc_base, ROWS_PER_TILE)],
                        cnt_out.at[pl.ds(acc_base, ROWS_PER_TILE)])


def _make_sc_counts():
    mesh = plsc.VectorSubcoreMesh(core_axis_name="c", subcore_axis_name="s")
    scratch = (
        pltpu.VMEM_SHARED((SEG, 16), jnp.float32),
        pltpu.VMEM((SB,), jnp.int32),
        pltpu.VMEM((SB,), jnp.int32),
        pltpu.VMEM((NG, G), jnp.int32),
        pltpu.VMEM((G, 16), jnp.float32),
        pltpu.VMEM((ZR, 16), jnp.float32),
    )
    return pl.kernel(
        _cnt_body, mesh=mesh,
        out_type=(jax.ShapeDtypeStruct((SEG, 16), jnp.float32),),
        scratch_types=scratch,
        compiler_params=pltpu.CompilerParams(use_tc_tiling_on_sc=False))


def _dense_body(final_avg, x_ref, acc_ref, cnt_ref, root_ref, w_ref, b_ref,
                x0_ref, out_ref):
    x = x_ref[...]
    out = jnp.dot(x, root_ref[...], preferred_element_type=jnp.float32)
    out += b_ref[...]
    inv = 1.0 / jnp.clip(cnt_ref[:, :, 0], 1.0, None)
    acc = acc_ref[...]
    pieces = [acc[ch, r] * inv[r][:, None]
              for r in range(N_REL) for ch in range(NCHUNK)]
    bcat = jnp.concatenate(pieces, axis=1)
    out += jnp.dot(bcat, w_ref[...], preferred_element_type=jnp.float32)
    if final_avg:
        out_ref[...] = (x0_ref[...] + x + out) * (1.0 / 3.0)
    else:
        out_ref[...] = out


def _dense_layer(x, acc, cnt, root, wstack, b, x0, final_avg):
    grid = (N_NODES // BN,)
    return pl.pallas_call(
        functools.partial(_dense_body, final_avg),
        grid=grid,
        in_specs=[
            pl.BlockSpec((BN, EMB), lambda i: (i, 0)),
            pl.BlockSpec((NCHUNK, N_REL, BN, CW), lambda i: (0, 0, i, 0)),
            pl.BlockSpec((N_REL, BN, 16), lambda i: (0, i, 0)),
            pl.BlockSpec((EMB, EMB), lambda i: (0, 0)),
            pl.BlockSpec((N_REL * EMB, EMB), lambda i: (0, 0)),
            pl.BlockSpec((1, EMB), lambda i: (0, 0)),
            pl.BlockSpec((BN, EMB), lambda i: (i, 0)),
        ],
        out_specs=pl.BlockSpec((BN, EMB), lambda i: (i, 0)),
        out_shape=jax.ShapeDtypeStruct((N_NODES, EMB), jnp.float32),
    )(x, acc, cnt, root, wstack, b, x0)


_sc_segsum = _make_sc_segsum()
_sc_counts = _make_sc_counts()


def kernel(edge_index_mp, edge_type, emb, w0, root0, b0, w1, root1, b1):
    src = edge_index_mp[0]
    dst = edge_index_mp[1]

    (cnt16,) = _sc_counts(dst, edge_type)
    cnt = cnt16.reshape(N_REL, N_NODES, 16)
    (acc0,) = _sc_segsum(src, dst, edge_type, emb.reshape(SEG, CW))
    x1 = _dense_layer(emb, acc0.reshape(NCHUNK, N_REL, N_NODES, CW), cnt,
                      root0, w0.reshape(N_REL * EMB, EMB),
                      b0.reshape(1, EMB), emb, False)
    (acc1,) = _sc_segsum(src, dst, edge_type, x1.reshape(SEG, CW))
    x2f = _dense_layer(x1, acc1.reshape(NCHUNK, N_REL, N_NODES, CW), cnt,
                       root1, w1.reshape(N_REL * EMB, EMB),
                       b1.reshape(1, EMB), emb, True)
    return x2f


# speedup vs baseline: 16.9701x; 1.1559x over previous
import functools

import jax
import jax.numpy as jnp
from jax import lax
from jax.experimental import pallas as pl
from jax.experimental.pallas import tpu as pltpu
from jax.experimental.pallas import tpu_sc as plsc

N_NODES = 10000
EMB = 128
N_REL = 4
N_EDGES = 320000
SEG = N_REL * N_NODES

CW = 32
NCHUNK = EMB // CW
G = 80
NG = 25
SB = G * NG
RING = 12
SLAG = 4
TILE_EDGES = N_EDGES // 16
NSUPER = TILE_EDGES // SB
CNT_TILE_EDGES = N_EDGES // 32
NSUPER_CNT = CNT_TILE_EDGES // SB
ROWS_PER_TILE = SEG // 16
ZR = 100

BN = 1000


def _sc_body(src_hbm, dst_hbm, et_hbm, x2d_hbm, acc_out,
             acc_s, srcv, dstv, etv, gidx, segb, rows, zbuf, gsem, ssem):
    c = lax.axis_index("c")
    s = lax.axis_index("s")
    tile_base = s * TILE_EDGES
    acc_base = s * ROWS_PER_TILE

    def zfill(i, carry):
        zbuf[i, pl.ds(0, 16)] = jnp.zeros((16,), jnp.float32)
        zbuf[i, pl.ds(16, 16)] = jnp.zeros((16,), jnp.float32)
        return carry
    lax.fori_loop(0, ZR, zfill, 0)

    for p in range(2):
        chunk = c * 2 + p

        def zero_body(i, carry):
            pltpu.sync_copy(zbuf, acc_s.at[pl.ds(acc_base + i * ZR, ZR)])
            return carry
        lax.fori_loop(0, ROWS_PER_TILE // ZR, zero_body, 0)
        plsc.subcore_barrier()

        def super_body(sb, carry):
            base = tile_base + sb * SB
            pltpu.make_async_copy(
                src_hbm.at[pl.ds(base, SB)], srcv, gsem).start()
            pltpu.make_async_copy(
                dst_hbm.at[pl.ds(base, SB)], dstv, gsem).start()
            pltpu.make_async_copy(
                et_hbm.at[pl.ds(base, SB)], etv, gsem).start()
            pltpu.make_async_copy(
                src_hbm.at[pl.ds(base, SB)], srcv, gsem).wait()
            pltpu.make_async_copy(
                dst_hbm.at[pl.ds(base, SB)], dstv, gsem).wait()
            pltpu.make_async_copy(
                et_hbm.at[pl.ds(base, SB)], etv, gsem).wait()

            def idx_body(g, carry2):
                o = g * G
                for k in range(G // 16):
                    sv = srcv[pl.ds(o + 16 * k, 16)]
                    dv = dstv[pl.ds(o + 16 * k, 16)]
                    ev = etv[pl.ds(o + 16 * k, 16)]
                    gidx[g, pl.ds(16 * k, 16)] = sv * NCHUNK + chunk
                    segb[g, pl.ds(16 * k, 16)] = ev * N_NODES + dv
                return carry2
            lax.fori_loop(0, NG, idx_body, 0)

            def fire(g, carry2):
                pltpu.make_async_copy(
                    x2d_hbm.at[gidx.at[g]], rows.at[g], gsem).start()
                return carry2
            lax.fori_loop(0, min(RING, NG), fire, 0)

            def pipe_body(g, carry2):
                slot = lax.rem(g, RING)
                pltpu.make_async_copy(
                    x2d_hbm.at[gidx.at[g]], rows.at[slot], gsem).wait()
                pltpu.async_copy(rows.at[slot], acc_s.at[segb.at[g]],
                                 ssem, add=True)

                @pl.when(g >= SLAG)
                def _():
                    h = g - SLAG
                    hslot = lax.rem(h, RING)
                    pltpu.make_async_copy(
                        rows.at[hslot], acc_s.at[segb.at[h]], ssem).wait()

                    @pl.when(h + RING < NG)
                    def _():
                        nxt = h + RING
                        pltpu.make_async_copy(
                            x2d_hbm.at[gidx.at[nxt]],
                            rows.at[lax.rem(nxt, RING)], gsem).start()
                return carry2
            lax.fori_loop(0, NG, pipe_body, 0)

            def retire(t, carry2):
                h = NG - SLAG + t
                pltpu.make_async_copy(
                    rows.at[lax.rem(h, RING)], acc_s.at[segb.at[h]],
                    ssem).wait()
                return carry2
            lax.fori_loop(0, SLAG, retire, 0)
            return carry
        lax.fori_loop(0, NSUPER, super_body, 0)

        plsc.subcore_barrier()
        pltpu.sync_copy(
            acc_s.at[pl.ds(acc_base, ROWS_PER_TILE)],
            acc_out.at[chunk, pl.ds(acc_base, ROWS_PER_TILE)])
        plsc.subcore_barrier()


def _make_sc_segsum():
    mesh = plsc.VectorSubcoreMesh(core_axis_name="c", subcore_axis_name="s")
    scratch = (
        pltpu.VMEM_SHARED((SEG, CW), jnp.float32),
        pltpu.VMEM((SB,), jnp.int32),
        pltpu.VMEM((SB,), jnp.int32),
        pltpu.VMEM((SB,), jnp.int32),
        pltpu.VMEM((NG, G), jnp.int32),
        pltpu.VMEM((NG, G), jnp.int32),
        pltpu.VMEM((RING, G, CW), jnp.float32),
        pltpu.VMEM((ZR, CW), jnp.float32),
        pltpu.SemaphoreType.DMA,
        pltpu.SemaphoreType.DMA,
    )
    return pl.kernel(
        _sc_body, mesh=mesh,
        out_type=(jax.ShapeDtypeStruct((NCHUNK, SEG, CW), jnp.float32),),
        scratch_types=scratch,
        compiler_params=pltpu.CompilerParams(use_tc_tiling_on_sc=False))


def _cnt_body(dst_hbm, et_hbm, cnt_out,
              cnt_s, dstv, etv, segb, ones, zbuf16):
    c = lax.axis_index("c")
    s = lax.axis_index("s")
    tile_base = (c * 16 + s) * CNT_TILE_EDGES
    acc_base = s * ROWS_PER_TILE

    def zfill(i, carry):
        zbuf16[i, pl.ds(0, 16)] = jnp.zeros((16,), jnp.float32)
        return carry
    lax.fori_loop(0, ZR, zfill, 0)

    def ofill(i, carry):
        ones[i, pl.ds(0, 16)] = jnp.ones((16,), jnp.float32)
        return carry
    lax.fori_loop(0, G, ofill, 0)

    def zero_body(i, carry):
        pltpu.sync_copy(zbuf16, cnt_s.at[pl.ds(acc_base + i * ZR, ZR)])
        return carry
    lax.fori_loop(0, ROWS_PER_TILE // ZR, zero_body, 0)
    plsc.subcore_barrier()

    def super_body(sb, carry):
        base = tile_base + sb * SB
        pltpu.sync_copy(dst_hbm.at[pl.ds(base, SB)], dstv)
        pltpu.sync_copy(et_hbm.at[pl.ds(base, SB)], etv)

        def idx_body(g, carry2):
            o = g * G
            for k in range(G // 16):
                dv = dstv[pl.ds(o + 16 * k, 16)]
                ev = etv[pl.ds(o + 16 * k, 16)]
                segb[g, pl.ds(16 * k, 16)] = ev * N_NODES + dv
            return carry2
        lax.fori_loop(0, NG, idx_body, 0)

        def scat_body(g, carry2):
            pltpu.sync_copy(ones, cnt_s.at[segb.at[g]], add=True)
            return carry2
        lax.fori_loop(0, NG, scat_body, 0)
        return carry
    lax.fori_loop(0, NSUPER_CNT, super_body, 0)
    plsc.subcore_barrier()

    pltpu.sync_copy(cnt_s.at[pl.ds(acc_base, ROWS_PER_TILE)],
                    cnt_out.at[c, pl.ds(acc_base, ROWS_PER_TILE)])


def _make_sc_counts():
    mesh = plsc.VectorSubcoreMesh(core_axis_name="c", subcore_axis_name="s")
    scratch = (
        pltpu.VMEM_SHARED((SEG, 16), jnp.float32),
        pltpu.VMEM((SB,), jnp.int32),
        pltpu.VMEM((SB,), jnp.int32),
        pltpu.VMEM((NG, G), jnp.int32),
        pltpu.VMEM((G, 16), jnp.float32),
        pltpu.VMEM((ZR, 16), jnp.float32),
    )
    return pl.kernel(
        _cnt_body, mesh=mesh,
        out_type=(jax.ShapeDtypeStruct((2, SEG, 16), jnp.float32),),
        scratch_types=scratch,
        compiler_params=pltpu.CompilerParams(use_tc_tiling_on_sc=False))


def _dense_body(final_avg, x_ref, acc_ref, cnt_ref, root_ref, w_ref, b_ref,
                x0_ref, out_ref):
    x = x_ref[...]
    out = jnp.dot(x, root_ref[...], preferred_element_type=jnp.float32)
    out += b_ref[...]
    cnt = cnt_ref[0, :, :, 0] + cnt_ref[1, :, :, 0]
    inv = 1.0 / jnp.clip(cnt, 1.0, None)
    acc = acc_ref[...]
    pieces = [acc[ch, r] * inv[r][:, None]
              for r in range(N_REL) for ch in range(NCHUNK)]
    bcat = jnp.concatenate(pieces, axis=1)
    out += jnp.dot(bcat, w_ref[...], preferred_element_type=jnp.float32)
    if final_avg:
        out_ref[...] = (x0_ref[...] + x + out) * (1.0 / 3.0)
    else:
        out_ref[...] = out


def _dense_layer(x, acc, cnt, root, wstack, b, x0, final_avg):
    grid = (N_NODES // BN,)
    return pl.pallas_call(
        functools.partial(_dense_body, final_avg),
        grid=grid,
        in_specs=[
            pl.BlockSpec((BN, EMB), lambda i: (i, 0)),
            pl.BlockSpec((NCHUNK, N_REL, BN, CW), lambda i: (0, 0, i, 0)),
            pl.BlockSpec((2, N_REL, BN, 16), lambda i: (0, 0, i, 0)),
            pl.BlockSpec((EMB, EMB), lambda i: (0, 0)),
            pl.BlockSpec((N_REL * EMB, EMB), lambda i: (0, 0)),
            pl.BlockSpec((1, EMB), lambda i: (0, 0)),
            pl.BlockSpec((BN, EMB), lambda i: (i, 0)),
        ],
        out_specs=pl.BlockSpec((BN, EMB), lambda i: (i, 0)),
        out_shape=jax.ShapeDtypeStruct((N_NODES, EMB), jnp.float32),
    )(x, acc, cnt, root, wstack, b, x0)


_sc_segsum = _make_sc_segsum()
_sc_counts = _make_sc_counts()


def kernel(edge_index_mp, edge_type, emb, w0, root0, b0, w1, root1, b1):
    src = edge_index_mp[0]
    dst = edge_index_mp[1]

    (cnt16,) = _sc_counts(dst, edge_type)
    cnt = cnt16.reshape(2, N_REL, N_NODES, 16)
    (acc0,) = _sc_segsum(src, dst, edge_type, emb.reshape(SEG, CW))
    x1 = _dense_layer(emb, acc0.reshape(NCHUNK, N_REL, N_NODES, CW), cnt,
                      root0, w0.reshape(N_REL * EMB, EMB),
                      b0.reshape(1, EMB), emb, False)
    (acc1,) = _sc_segsum(src, dst, edge_type, x1.reshape(SEG, CW))
    x2f = _dense_layer(x1, acc1.reshape(NCHUNK, N_REL, N_NODES, CW), cnt,
                       root1, w1.reshape(N_REL * EMB, EMB),
                       b1.reshape(1, EMB), emb, True)
    return x2f


# speedup vs baseline: 23.5869x; 1.3899x over previous
import functools

import jax
import jax.numpy as jnp
from jax import lax
from jax.experimental import pallas as pl
from jax.experimental.pallas import tpu as pltpu
from jax.experimental.pallas import tpu_sc as plsc

N_NODES = 10000
EMB = 128
N_REL = 4
N_EDGES = 320000
SEG = N_REL * N_NODES

CW = 32
NCHUNK = EMB // CW
G = 80
NG = 25
SB = G * NG
RING = 12
SLAG = 4
TILE_EDGES = N_EDGES // 16
NSUPER = TILE_EDGES // SB
CNT_TILE_EDGES = N_EDGES // 32
NSUPER_CNT = CNT_TILE_EDGES // SB
ROWS_PER_TILE = SEG // 16
ZR = 100

BN = 1000


def _sc_body(src_hbm, dst_hbm, et_hbm, x2d_hbm, acc_out,
             acc_s, srcv, dstv, etv, gidx, segb, rows, zbuf, gsem, ssem):
    c = lax.axis_index("c")
    s = lax.axis_index("s")
    tile_base = s * TILE_EDGES
    acc_base = s * ROWS_PER_TILE

    def zfill(i, carry):
        zbuf[i, pl.ds(0, 16)] = jnp.zeros((16,), jnp.float32)
        zbuf[i, pl.ds(16, 16)] = jnp.zeros((16,), jnp.float32)
        return carry
    lax.fori_loop(0, ZR, zfill, 0)

    for p in range(2):
        chunk = c * 2 + p

        def zero_body(i, carry):
            pltpu.sync_copy(zbuf, acc_s.at[pl.ds(acc_base + i * ZR, ZR)])
            return carry
        lax.fori_loop(0, ROWS_PER_TILE // ZR, zero_body, 0)
        plsc.subcore_barrier()

        def super_body(sb, carry):
            base = tile_base + sb * SB
            pltpu.make_async_copy(
                src_hbm.at[pl.ds(base, SB)], srcv, gsem).start()
            pltpu.make_async_copy(
                dst_hbm.at[pl.ds(base, SB)], dstv, gsem).start()
            pltpu.make_async_copy(
                et_hbm.at[pl.ds(base, SB)], etv, gsem).start()
            pltpu.make_async_copy(
                src_hbm.at[pl.ds(base, SB)], srcv, gsem).wait()
            pltpu.make_async_copy(
                dst_hbm.at[pl.ds(base, SB)], dstv, gsem).wait()
            pltpu.make_async_copy(
                et_hbm.at[pl.ds(base, SB)], etv, gsem).wait()

            def idx_body(g, carry2):
                o = g * G
                for k in range(G // 16):
                    sv = srcv[pl.ds(o + 16 * k, 16)]
                    dv = dstv[pl.ds(o + 16 * k, 16)]
                    ev = etv[pl.ds(o + 16 * k, 16)]
                    gidx[g, pl.ds(16 * k, 16)] = sv * NCHUNK + chunk
                    segb[g, pl.ds(16 * k, 16)] = dv * N_REL + ev
                return carry2
            lax.fori_loop(0, NG, idx_body, 0)

            def fire(g, carry2):
                pltpu.make_async_copy(
                    x2d_hbm.at[gidx.at[g]], rows.at[g], gsem).start()
                return carry2
            lax.fori_loop(0, min(RING, NG), fire, 0)

            def pipe_body(g, carry2):
                slot = lax.rem(g, RING)
                pltpu.make_async_copy(
                    x2d_hbm.at[gidx.at[g]], rows.at[slot], gsem).wait()
                pltpu.async_copy(rows.at[slot], acc_s.at[segb.at[g]],
                                 ssem, add=True)

                @pl.when(g >= SLAG)
                def _():
                    h = g - SLAG
                    hslot = lax.rem(h, RING)
                    pltpu.make_async_copy(
                        rows.at[hslot], acc_s.at[segb.at[h]], ssem).wait()

                    @pl.when(h + RING < NG)
                    def _():
                        nxt = h + RING
                        pltpu.make_async_copy(
                            x2d_hbm.at[gidx.at[nxt]],
                            rows.at[lax.rem(nxt, RING)], gsem).start()
                return carry2
            lax.fori_loop(0, NG, pipe_body, 0)

            def retire(t, carry2):
                h = NG - SLAG + t
                pltpu.make_async_copy(
                    rows.at[lax.rem(h, RING)], acc_s.at[segb.at[h]],
                    ssem).wait()
                return carry2
            lax.fori_loop(0, SLAG, retire, 0)
            return carry
        lax.fori_loop(0, NSUPER, super_body, 0)

        plsc.subcore_barrier()
        pltpu.sync_copy(
            acc_s.at[pl.ds(acc_base, ROWS_PER_TILE)],
            acc_out.at[chunk, pl.ds(acc_base, ROWS_PER_TILE)])
        plsc.subcore_barrier()


def _make_sc_segsum():
    mesh = plsc.VectorSubcoreMesh(core_axis_name="c", subcore_axis_name="s")
    scratch = (
        pltpu.VMEM_SHARED((SEG, CW), jnp.float32),
        pltpu.VMEM((SB,), jnp.int32),
        pltpu.VMEM((SB,), jnp.int32),
        pltpu.VMEM((SB,), jnp.int32),
        pltpu.VMEM((NG, G), jnp.int32),
        pltpu.VMEM((NG, G), jnp.int32),
        pltpu.VMEM((RING, G, CW), jnp.float32),
        pltpu.VMEM((ZR, CW), jnp.float32),
        pltpu.SemaphoreType.DMA,
        pltpu.SemaphoreType.DMA,
    )
    return pl.kernel(
        _sc_body, mesh=mesh,
        out_type=(jax.ShapeDtypeStruct((NCHUNK, SEG, CW), jnp.float32),),
        scratch_types=scratch,
        compiler_params=pltpu.CompilerParams(use_tc_tiling_on_sc=False))


def _cnt_body(dst_hbm, et_hbm, cnt_out,
              cnt_s, dstv, etv, segb, ones, zbuf16):
    c = lax.axis_index("c")
    s = lax.axis_index("s")
    tile_base = (c * 16 + s) * CNT_TILE_EDGES
    acc_base = s * ROWS_PER_TILE

    def zfill(i, carry):
        zbuf16[i, pl.ds(0, 16)] = jnp.zeros((16,), jnp.float32)
        return carry
    lax.fori_loop(0, ZR, zfill, 0)

    def ofill(i, carry):
        ones[i, pl.ds(0, 16)] = jnp.ones((16,), jnp.float32)
        return carry
    lax.fori_loop(0, G, ofill, 0)

    def zero_body(i, carry):
        pltpu.sync_copy(zbuf16, cnt_s.at[pl.ds(acc_base + i * ZR, ZR)])
        return carry
    lax.fori_loop(0, ROWS_PER_TILE // ZR, zero_body, 0)
    plsc.subcore_barrier()

    def super_body(sb, carry):
        base = tile_base + sb * SB
        pltpu.sync_copy(dst_hbm.at[pl.ds(base, SB)], dstv)
        pltpu.sync_copy(et_hbm.at[pl.ds(base, SB)], etv)

        def idx_body(g, carry2):
            o = g * G
            for k in range(G // 16):
                dv = dstv[pl.ds(o + 16 * k, 16)]
                ev = etv[pl.ds(o + 16 * k, 16)]
                segb[g, pl.ds(16 * k, 16)] = dv * N_REL + ev
            return carry2
        lax.fori_loop(0, NG, idx_body, 0)

        def scat_body(g, carry2):
            pltpu.sync_copy(ones, cnt_s.at[segb.at[g]], add=True)
            return carry2
        lax.fori_loop(0, NG, scat_body, 0)
        return carry
    lax.fori_loop(0, NSUPER_CNT, super_body, 0)
    plsc.subcore_barrier()

    pltpu.sync_copy(cnt_s.at[pl.ds(acc_base, ROWS_PER_TILE)],
                    cnt_out.at[c, pl.ds(acc_base, ROWS_PER_TILE)])


def _make_sc_counts():
    mesh = plsc.VectorSubcoreMesh(core_axis_name="c", subcore_axis_name="s")
    scratch = (
        pltpu.VMEM_SHARED((SEG, 16), jnp.float32),
        pltpu.VMEM((SB,), jnp.int32),
        pltpu.VMEM((SB,), jnp.int32),
        pltpu.VMEM((NG, G), jnp.int32),
        pltpu.VMEM((G, 16), jnp.float32),
        pltpu.VMEM((ZR, 16), jnp.float32),
    )
    return pl.kernel(
        _cnt_body, mesh=mesh,
        out_type=(jax.ShapeDtypeStruct((2, SEG, 16), jnp.float32),),
        scratch_types=scratch,
        compiler_params=pltpu.CompilerParams(use_tc_tiling_on_sc=False))


def _dense_body(final_avg, x_ref, acc_ref, inv_ref, root_ref, w_ref, b_ref,
                x0_ref, out_ref):
    x = x_ref[...]
    out = jnp.dot(x, root_ref[...], preferred_element_type=jnp.float32)
    out += b_ref[...]
    inv = inv_ref[...]
    scaled = [acc_ref[ch] * inv for ch in range(NCHUNK)]
    pieces = [scaled[ch][:, r * CW:(r + 1) * CW]
              for r in range(N_REL) for ch in range(NCHUNK)]
    bcat = jnp.concatenate(pieces, axis=1)
    out += jnp.dot(bcat, w_ref[...], preferred_element_type=jnp.float32)
    if final_avg:
        out_ref[...] = (x0_ref[...] + x + out) * (1.0 / 3.0)
    else:
        out_ref[...] = out


def _dense_layer(x, acc_t, inv_t, root, wstack, b, x0, final_avg):
    grid = (N_NODES // BN,)
    return pl.pallas_call(
        functools.partial(_dense_body, final_avg),
        grid=grid,
        in_specs=[
            pl.BlockSpec((BN, EMB), lambda i: (i, 0)),
            pl.BlockSpec((NCHUNK, BN, EMB), lambda i: (0, i, 0)),
            pl.BlockSpec((BN, EMB), lambda i: (i, 0)),
            pl.BlockSpec((EMB, EMB), lambda i: (0, 0)),
            pl.BlockSpec((N_REL * EMB, EMB), lambda i: (0, 0)),
            pl.BlockSpec((1, EMB), lambda i: (0, 0)),
            pl.BlockSpec((BN, EMB), lambda i: (i, 0)),
        ],
        out_specs=pl.BlockSpec((BN, EMB), lambda i: (i, 0)),
        out_shape=jax.ShapeDtypeStruct((N_NODES, EMB), jnp.float32),
    )(x, acc_t, inv_t, root, wstack, b, x0)


_sc_segsum = _make_sc_segsum()
_sc_counts = _make_sc_counts()


def kernel(edge_index_mp, edge_type, emb, w0, root0, b0, w1, root1, b1):
    src = edge_index_mp[0]
    dst = edge_index_mp[1]

    (cnt16,) = _sc_counts(dst, edge_type)
    cnts = cnt16[0, :, 0] + cnt16[1, :, 0]
    inv = 1.0 / jnp.clip(cnts.reshape(N_NODES, N_REL), 1.0, None)
    inv_t = jnp.repeat(inv, CW, axis=1)

    (acc0,) = _sc_segsum(src, dst, edge_type, emb.reshape(SEG, CW))
    x1 = _dense_layer(emb, acc0.reshape(NCHUNK, N_NODES, EMB), inv_t,
                      root0, w0.reshape(N_REL * EMB, EMB),
                      b0.reshape(1, EMB), emb, False)
    (acc1,) = _sc_segsum(src, dst, edge_type, x1.reshape(SEG, CW))
    x2f = _dense_layer(x1, acc1.reshape(NCHUNK, N_NODES, EMB), inv_t,
                       root1, w1.reshape(N_REL * EMB, EMB),
                       b1.reshape(1, EMB), emb, True)
    return x2f
